# baseline (speedup 1.0000x reference)
import functools

import jax
import jax.numpy as jnp
from jax import lax
from jax.experimental import pallas as pl
from jax.experimental.pallas import tpu as pltpu
from jax.experimental.pallas import tpu_sc as plsc

NC = 2
NS = 16
NW = NC * NS
CHUNK = 64
GRP = 8
SPLIT0_PCT = 0


def _node_matmul(nf, W, Wb):
    n, d = nf.shape
    d2 = W.shape[1]
    blk = 2000
    grid = n // blk

    def body(nf_ref, w_ref, b_ref, hs_ref, h_ref):
        hall = jnp.dot(nf_ref[...], w_ref[...],
                       preferred_element_type=jnp.float32) + b_ref[...]
        hs_ref[...] = hall[:, : d2 // 2]
        h_ref[...] = hall[:, d2 // 2:]

    return pl.pallas_call(
        body,
        grid=(grid,),
        in_specs=[
            pl.BlockSpec((blk, d), lambda i: (i, 0)),
            pl.BlockSpec((d, d2), lambda i: (0, 0)),
            pl.BlockSpec((1, d2), lambda i: (0, 0)),
        ],
        out_specs=[
            pl.BlockSpec((blk, d2 // 2), lambda i: (i, 0)),
            pl.BlockSpec((blk, d2 // 2), lambda i: (i, 0)),
        ],
        out_shape=[
            jax.ShapeDtypeStruct((n, d2 // 2), jnp.float32),
            jax.ShapeDtypeStruct((n, d2 // 2), jnp.float32),
        ],
    )(nf, W, Wb.reshape(1, d2))


def _sc_aggregate(h, send2d, recv2d, eidx, ef1, zA, zD, zE, R, cpt0, cpt1):
    D = h.shape[1]
    DE = 16
    rpt = R // NS
    mesh = plsc.VectorSubcoreMesh(core_axis_name="c", subcore_axis_name="s")

    @functools.partial(
        pl.kernel,
        mesh=mesh,
        out_type=[
            jax.ShapeDtypeStruct((NC * R, D), jnp.float32),
            jax.ShapeDtypeStruct((NC * R * DE,), jnp.float32),
            jax.ShapeDtypeStruct((NC * R,), jnp.float32),
        ],
        scratch_types=[
            pltpu.VMEM((GRP, CHUNK), jnp.int32),
            pltpu.VMEM((GRP, CHUNK), jnp.int32),
            pltpu.VMEM((2, CHUNK, D), jnp.float32),
            pltpu.VMEM((2, CHUNK * DE // 128, 128), jnp.int32),
            pltpu.VMEM((2, CHUNK * DE), jnp.float32),
            pltpu.VMEM((CHUNK,), jnp.float32),
            pltpu.VMEM_SHARED((R, D), jnp.float32),
            pltpu.VMEM_SHARED((R * DE,), jnp.float32),
            pltpu.VMEM_SHARED((R,), jnp.float32),
            pltpu.SemaphoreType.DMA,
            pltpu.SemaphoreType.DMA,
            pltpu.SemaphoreType.DMA,
            pltpu.SemaphoreType.DMA,
            pltpu.SemaphoreType.DMA,
            pltpu.SemaphoreType.DMA,
            pltpu.SemaphoreType.DMA,
            pltpu.SemaphoreType.DMA,
        ],
    )
    def k(h_hbm, s_hbm, r_hbm, eidx_hbm, ef1_hbm, zA_hbm, zD_hbm, zE_hbm,
          outH, outEF, outD,
          idx_s, idx_r, hbuf, eidx_v, efb1, ones64, accH, accEF, accD,
          sg0, sg1, sx0, sx1, sf0, sf1, ss0, ss1):
        c = lax.axis_index("c")
        s = lax.axis_index("s")
        w = c * NS + s
        nsub = CHUNK * DE // 128
        sg = (sg0, sg1)
        sx = (sx0, sx1)
        sf = (sf0, sf1)
        ss = (ss0, ss1)

        pltpu.sync_copy(zA_hbm, accH.at[pl.ds(s * rpt, rpt)])
        pltpu.sync_copy(zD_hbm, accD.at[pl.ds(s * rpt, rpt)])
        pltpu.sync_copy(zE_hbm, accEF.at[pl.ds(s * rpt * DE, rpt * DE)])

        def ob(u, carry):
            ones64[pl.ds(u * 16, 16)] = jnp.ones((16,), jnp.float32)
            return carry

        lax.fori_loop(0, CHUNK // 16, ob, 0)
        plsc.subcore_barrier()

        def loads(g0, j, b):
            j0 = g0 + j
            return (
                pltpu.make_async_copy(h_hbm.at[idx_s.at[j]],
                                      hbuf.at[b], sg[b]),
                pltpu.make_async_copy(eidx_hbm.at[pl.ds(j0 * nsub, nsub)],
                                      eidx_v.at[b], sx[b]),
                pltpu.make_async_copy(
                    ef1_hbm.at[pl.ds(j0 * CHUNK * DE, CHUNK * DE)],
                    efb1.at[b], sf[b]),
            )

        def start(g0, j, b):
            for cp in loads(g0, j, b):
                cp.start()

        def scatter_start(j, b):
            pltpu.async_copy(hbuf.at[b], accH.at[idx_r.at[j]], ss[b],
                             add=True)
            for u in range(nsub):
                pltpu.async_copy(efb1.at[b].at[pl.ds(u * 128, 128)],
                                 accEF.at[eidx_v.at[b].at[u]], ss[b],
                                 add=True)
            pltpu.async_copy(ones64, accD.at[idx_r.at[j]], ss[b], add=True)

        def scatter_drain(b):
            pltpu.make_async_copy(hbuf.at[b], accH.at[idx_r.at[0]],
                                  ss[b]).wait()
            pltpu.make_async_copy(efb1.at[b], accEF.at[pl.ds(0, CHUNK * DE)],
                                  ss[b]).wait()
            pltpu.make_async_copy(ones64, accD.at[pl.ds(0, CHUNK)],
                                  ss[b]).wait()

        base = jnp.where(c == 0, s * cpt0, NS * cpt0 + s * cpt1)
        ngrp = jnp.where(c == 0, cpt0 // GRP, cpt1 // GRP)

        def group(g, carry):
            g0 = base + g * GRP
            pltpu.sync_copy(s_hbm.at[pl.ds(g0, GRP)], idx_s)
            pltpu.sync_copy(r_hbm.at[pl.ds(g0, GRP)], idx_r)
            start(g0, 0, 0)

            def pair(p, carry2):
                for b in range(2):
                    j = 2 * p + b
                    for cp in loads(g0, j, b):
                        cp.wait()
                    scatter_start(j, b)
                    if b == 0:
                        @pl.when((g > 0) | (p > 0))
                        def _():
                            scatter_drain(1)
                    else:
                        scatter_drain(0)

                    @pl.when(j < GRP - 1)
                    def _():
                        start(g0, j + 1, 1 - b)
                return carry2

            lax.fori_loop(0, GRP // 2, pair, 0)
            return carry

        lax.fori_loop(0, ngrp, group, 0)

        @pl.when(ngrp > 0)
        def _():
            scatter_drain(1)

        plsc.subcore_barrier()

        r0 = s * rpt
        pltpu.sync_copy(accH.at[pl.ds(r0, rpt)],
                        outH.at[pl.ds(c * R + r0, rpt)])
        pltpu.sync_copy(accEF.at[pl.ds(r0 * DE, rpt * DE)],
                        outEF.at[pl.ds((c * R + r0) * DE, rpt * DE)])
        pltpu.sync_copy(accD.at[pl.ds(r0, rpt)],
                        outD.at[pl.ds(c * R + r0, rpt)])

    return k(h, send2d, recv2d, eidx, ef1, zA, zD, zE)


def _combine(h_self, aggH, aggEF, aggD, We, Web):
    n, d = h_self.shape
    de = We.shape[0]
    blk = 2000
    grid = n // blk

    def body(hs_ref, aH_ref, aE_ref, aD_ref, we_ref, wb_ref, out_ref):
        deg = aD_ref[0] + aD_ref[1]
        aggh = aH_ref[0] + aH_ref[1]
        aggef = aE_ref[0] + aE_ref[1]
        t = aggh + jnp.dot(aggef, we_ref[...],
                           preferred_element_type=jnp.float32)
        t = t + deg * wb_ref[...]
        out_ref[...] = hs_ref[...] + jnp.maximum(deg, 1.0) * t

    return pl.pallas_call(
        body,
        grid=(grid,),
        in_specs=[
            pl.BlockSpec((blk, d), lambda i: (i, 0)),
            pl.BlockSpec((2, blk, d), lambda i: (0, i, 0)),
            pl.BlockSpec((2, blk, de), lambda i: (0, i, 0)),
            pl.BlockSpec((2, blk, 1), lambda i: (0, i, 0)),
            pl.BlockSpec((de, d), lambda i: (0, 0)),
            pl.BlockSpec((1, d), lambda i: (0, 0)),
        ],
        out_specs=pl.BlockSpec((blk, d), lambda i: (i, 0)),
        out_shape=jax.ShapeDtypeStruct((n, d), jnp.float32),
    )(h_self, aggH, aggEF, aggD, We, Web.reshape(1, d))


@jax.jit
def kernel(node_features, senders, receivers, edge_features,
           W_kernel, W_bias, We_kernel, We_bias):
    n, d = node_features.shape
    e = senders.shape[0]
    de = edge_features.shape[1]

    cpt = -(-(-(-e // (NW * CHUNK))) // GRP) * GRP
    tot = 2 * cpt
    cpt0 = -(-(tot * SPLIT0_PCT // 100) // GRP) * GRP
    cpt1 = tot - cpt0
    e_pad = NS * tot * CHUNK
    pad = e_pad - e
    R = -(-(n + 1) // (NS * 64)) * (NS * 64)
    pad_recv = n + jnp.arange(pad, dtype=jnp.int32) % (R - n)
    s32 = jnp.concatenate(
        [senders.astype(jnp.int32), jnp.zeros((pad,), jnp.int32)])
    r32 = jnp.concatenate([receivers.astype(jnp.int32), pad_recv])
    send2d = s32.reshape(e_pad // CHUNK, CHUNK)
    recv2d = r32.reshape(e_pad // CHUNK, CHUNK)
    eidx = (r32[:, None] * de +
            jnp.arange(de, dtype=jnp.int32)[None, :]).reshape(-1, 128)
    ef1 = jnp.concatenate(
        [edge_features.reshape(-1), jnp.zeros((pad * de,), jnp.float32)])

    rpt = R // NS
    zA = jnp.zeros((rpt, d), jnp.float32)
    zD = jnp.zeros((rpt,), jnp.float32)
    zE = jnp.zeros((rpt * de,), jnp.float32)

    h_self, h = _node_matmul(node_features, W_kernel, W_bias)
    outH, outEF, outD = _sc_aggregate(
        h, send2d, recv2d, eidx, ef1, zA, zD, zE, R, cpt0, cpt1)
    aggH = outH.reshape(NC, R, d)
    aggEF = outEF.reshape(NC, R, de)
    aggD = outD.reshape(NC, R, 1)
    return _combine(h_self, aggH, aggEF, aggD, We_kernel, We_bias)


# speedup vs baseline: 1.2015x; 1.2015x over previous
import functools

import jax
import jax.numpy as jnp
from jax import lax
from jax.experimental import pallas as pl
from jax.experimental.pallas import tpu as pltpu
from jax.experimental.pallas import tpu_sc as plsc

NC = 2
NS = 16
NW = NC * NS
CHUNK = 64
GRP = 8
SPLIT0_PCT = 50


def _node_matmul(nf, W, Wb):
    n, d = nf.shape
    d2 = W.shape[1]
    blk = 2000
    grid = n // blk

    def body(nf_ref, w_ref, b_ref, hs_ref, h_ref):
        hall = jnp.dot(nf_ref[...], w_ref[...],
                       preferred_element_type=jnp.float32) + b_ref[...]
        hs_ref[...] = hall[:, : d2 // 2]
        h_ref[...] = hall[:, d2 // 2:]

    return pl.pallas_call(
        body,
        grid=(grid,),
        in_specs=[
            pl.BlockSpec((blk, d), lambda i: (i, 0)),
            pl.BlockSpec((d, d2), lambda i: (0, 0)),
            pl.BlockSpec((1, d2), lambda i: (0, 0)),
        ],
        out_specs=[
            pl.BlockSpec((blk, d2 // 2), lambda i: (i, 0)),
            pl.BlockSpec((blk, d2 // 2), lambda i: (i, 0)),
        ],
        out_shape=[
            jax.ShapeDtypeStruct((n, d2 // 2), jnp.float32),
            jax.ShapeDtypeStruct((n, d2 // 2), jnp.float32),
        ],
    )(nf, W, Wb.reshape(1, d2))


def _sc_aggregate(h, send2d, recv2d, eidx, ef1, zA, zD, zE, R, cpt0, cpt1):
    D = h.shape[1]
    DE = 16
    rpt = R // NS
    mesh = plsc.VectorSubcoreMesh(core_axis_name="c", subcore_axis_name="s")

    @functools.partial(
        pl.kernel,
        mesh=mesh,
        out_type=[
            jax.ShapeDtypeStruct((NC * R, D), jnp.float32),
            jax.ShapeDtypeStruct((NC * R * DE,), jnp.float32),
            jax.ShapeDtypeStruct((NC * R,), jnp.float32),
        ],
        scratch_types=[
            pltpu.VMEM((GRP, CHUNK), jnp.int32),
            pltpu.VMEM((GRP, CHUNK), jnp.int32),
            pltpu.VMEM((2, CHUNK, D), jnp.float32),
            pltpu.VMEM((2, CHUNK * DE // 128, 128), jnp.int32),
            pltpu.VMEM((2, CHUNK * DE), jnp.float32),
            pltpu.VMEM((CHUNK,), jnp.float32),
            pltpu.VMEM_SHARED((R, D), jnp.float32),
            pltpu.VMEM_SHARED((R * DE,), jnp.float32),
            pltpu.VMEM_SHARED((R,), jnp.float32),
            pltpu.SemaphoreType.DMA,
            pltpu.SemaphoreType.DMA,
            pltpu.SemaphoreType.DMA,
            pltpu.SemaphoreType.DMA,
            pltpu.SemaphoreType.DMA,
            pltpu.SemaphoreType.DMA,
            pltpu.SemaphoreType.DMA,
            pltpu.SemaphoreType.DMA,
        ],
    )
    def k(h_hbm, s_hbm, r_hbm, eidx_hbm, ef1_hbm, zA_hbm, zD_hbm, zE_hbm,
          outH, outEF, outD,
          idx_s, idx_r, hbuf, eidx_v, efb1, ones64, accH, accEF, accD,
          sg0, sg1, sx0, sx1, sf0, sf1, ss0, ss1):
        c = lax.axis_index("c")
        s = lax.axis_index("s")
        w = c * NS + s
        nsub = CHUNK * DE // 128
        sg = (sg0, sg1)
        sx = (sx0, sx1)
        sf = (sf0, sf1)
        ss = (ss0, ss1)

        pltpu.sync_copy(zA_hbm, accH.at[pl.ds(s * rpt, rpt)])
        pltpu.sync_copy(zD_hbm, accD.at[pl.ds(s * rpt, rpt)])
        pltpu.sync_copy(zE_hbm, accEF.at[pl.ds(s * rpt * DE, rpt * DE)])

        def ob(u, carry):
            ones64[pl.ds(u * 16, 16)] = jnp.ones((16,), jnp.float32)
            return carry

        lax.fori_loop(0, CHUNK // 16, ob, 0)
        plsc.subcore_barrier()

        def loads(g0, j, b):
            j0 = g0 + j
            return (
                pltpu.make_async_copy(h_hbm.at[idx_s.at[j]],
                                      hbuf.at[b], sg[b]),
                pltpu.make_async_copy(eidx_hbm.at[pl.ds(j0 * nsub, nsub)],
                                      eidx_v.at[b], sx[b]),
                pltpu.make_async_copy(
                    ef1_hbm.at[pl.ds(j0 * CHUNK * DE, CHUNK * DE)],
                    efb1.at[b], sf[b]),
            )

        def start(g0, j, b):
            for cp in loads(g0, j, b):
                cp.start()

        def scatter_start(j, b):
            pltpu.async_copy(hbuf.at[b], accH.at[idx_r.at[j]], ss[b],
                             add=True)
            for u in range(nsub):
                pltpu.async_copy(efb1.at[b].at[pl.ds(u * 128, 128)],
                                 accEF.at[eidx_v.at[b].at[u]], ss[b],
                                 add=True)
            pltpu.async_copy(ones64, accD.at[idx_r.at[j]], ss[b], add=True)

        def scatter_drain(b):
            pltpu.make_async_copy(hbuf.at[b], accH.at[idx_r.at[0]],
                                  ss[b]).wait()
            pltpu.make_async_copy(efb1.at[b], accEF.at[pl.ds(0, CHUNK * DE)],
                                  ss[b]).wait()
            pltpu.make_async_copy(ones64, accD.at[pl.ds(0, CHUNK)],
                                  ss[b]).wait()

        base = jnp.where(c == 0, s * cpt0, NS * cpt0 + s * cpt1)
        ngrp = jnp.where(c == 0, cpt0 // GRP, cpt1 // GRP)

        def group(g, carry):
            g0 = base + g * GRP
            pltpu.sync_copy(s_hbm.at[pl.ds(g0, GRP)], idx_s)
            pltpu.sync_copy(r_hbm.at[pl.ds(g0, GRP)], idx_r)
            start(g0, 0, 0)

            def pair(p, carry2):
                for b in range(2):
                    j = 2 * p + b
                    for cp in loads(g0, j, b):
                        cp.wait()
                    scatter_start(j, b)
                    if b == 0:
                        @pl.when((g > 0) | (p > 0))
                        def _():
                            scatter_drain(1)
                    else:
                        scatter_drain(0)

                    @pl.when(j < GRP - 1)
                    def _():
                        start(g0, j + 1, 1 - b)
                return carry2

            lax.fori_loop(0, GRP // 2, pair, 0)
            return carry

        lax.fori_loop(0, ngrp, group, 0)

        @pl.when(ngrp > 0)
        def _():
            scatter_drain(1)

        plsc.subcore_barrier()

        r0 = s * rpt
        pltpu.sync_copy(accH.at[pl.ds(r0, rpt)],
                        outH.at[pl.ds(c * R + r0, rpt)])
        pltpu.sync_copy(accEF.at[pl.ds(r0 * DE, rpt * DE)],
                        outEF.at[pl.ds((c * R + r0) * DE, rpt * DE)])
        pltpu.sync_copy(accD.at[pl.ds(r0, rpt)],
                        outD.at[pl.ds(c * R + r0, rpt)])

    return k(h, send2d, recv2d, eidx, ef1, zA, zD, zE)


def _combine(h_self, aggH, aggEF, aggD, We, Web):
    n, d = h_self.shape
    de = We.shape[0]
    blk = 2000
    grid = n // blk

    def body(hs_ref, aH_ref, aE_ref, aD_ref, we_ref, wb_ref, out_ref):
        deg = aD_ref[0] + aD_ref[1]
        aggh = aH_ref[0] + aH_ref[1]
        aggef = aE_ref[0] + aE_ref[1]
        t = aggh + jnp.dot(aggef, we_ref[...],
                           preferred_element_type=jnp.float32)
        t = t + deg * wb_ref[...]
        out_ref[...] = hs_ref[...] + jnp.maximum(deg, 1.0) * t

    return pl.pallas_call(
        body,
        grid=(grid,),
        in_specs=[
            pl.BlockSpec((blk, d), lambda i: (i, 0)),
            pl.BlockSpec((2, blk, d), lambda i: (0, i, 0)),
            pl.BlockSpec((2, blk, de), lambda i: (0, i, 0)),
            pl.BlockSpec((2, blk, 1), lambda i: (0, i, 0)),
            pl.BlockSpec((de, d), lambda i: (0, 0)),
            pl.BlockSpec((1, d), lambda i: (0, 0)),
        ],
        out_specs=pl.BlockSpec((blk, d), lambda i: (i, 0)),
        out_shape=jax.ShapeDtypeStruct((n, d), jnp.float32),
    )(h_self, aggH, aggEF, aggD, We, Web.reshape(1, d))


@jax.jit
def kernel(node_features, senders, receivers, edge_features,
           W_kernel, W_bias, We_kernel, We_bias):
    n, d = node_features.shape
    e = senders.shape[0]
    de = edge_features.shape[1]

    cpt = -(-(-(-e // (NW * CHUNK))) // GRP) * GRP
    tot = 2 * cpt
    cpt0 = -(-(tot * SPLIT0_PCT // 100) // GRP) * GRP
    cpt1 = tot - cpt0
    e_pad = NS * tot * CHUNK
    pad = e_pad - e
    R = -(-(n + 1) // (NS * 64)) * (NS * 64)
    pad_recv = n + jnp.arange(pad, dtype=jnp.int32) % (R - n)
    s32 = jnp.concatenate(
        [senders.astype(jnp.int32), jnp.zeros((pad,), jnp.int32)])
    r32 = jnp.concatenate([receivers.astype(jnp.int32), pad_recv])
    send2d = s32.reshape(e_pad // CHUNK, CHUNK)
    recv2d = r32.reshape(e_pad // CHUNK, CHUNK)
    eidx = (r32[:, None] * de +
            jnp.arange(de, dtype=jnp.int32)[None, :]).reshape(-1, 128)
    ef1 = jnp.concatenate(
        [edge_features.reshape(-1), jnp.zeros((pad * de,), jnp.float32)])

    rpt = R // NS
    zA = jnp.zeros((rpt, d), jnp.float32)
    zD = jnp.zeros((rpt,), jnp.float32)
    zE = jnp.zeros((rpt * de,), jnp.float32)

    h_self, h = _node_matmul(node_features, W_kernel, W_bias)
    outH, outEF, outD = _sc_aggregate(
        h, send2d, recv2d, eidx, ef1, zA, zD, zE, R, cpt0, cpt1)
    aggH = outH.reshape(NC, R, d)
    aggEF = outEF.reshape(NC, R, de)
    aggD = outD.reshape(NC, R, 1)
    return _combine(h_self, aggH, aggEF, aggD, We_kernel, We_bias)


# speedup vs baseline: 1.2438x; 1.0352x over previous
import functools

import jax
import jax.numpy as jnp
from jax import lax
from jax.experimental import pallas as pl
from jax.experimental.pallas import tpu as pltpu
from jax.experimental.pallas import tpu_sc as plsc

NC = 2
NS = 16
NW = NC * NS
CHUNK = 64
GRP = 8


def _node_matmul(nf, W, Wb):
    n, d = nf.shape
    d2 = W.shape[1]
    blk = 2000
    grid = n // blk

    def body(nf_ref, w_ref, b_ref, hs_ref, h_ref):
        hall = jnp.dot(nf_ref[...], w_ref[...],
                       preferred_element_type=jnp.float32) + b_ref[...]
        hs_ref[...] = hall[:, : d2 // 2]
        h_ref[...] = hall[:, d2 // 2:]

    return pl.pallas_call(
        body,
        grid=(grid,),
        in_specs=[
            pl.BlockSpec((blk, d), lambda i: (i, 0)),
            pl.BlockSpec((d, d2), lambda i: (0, 0)),
            pl.BlockSpec((1, d2), lambda i: (0, 0)),
        ],
        out_specs=[
            pl.BlockSpec((blk, d2 // 2), lambda i: (i, 0)),
            pl.BlockSpec((blk, d2 // 2), lambda i: (i, 0)),
        ],
        out_shape=[
            jax.ShapeDtypeStruct((n, d2 // 2), jnp.float32),
            jax.ShapeDtypeStruct((n, d2 // 2), jnp.float32),
        ],
    )(nf, W, Wb.reshape(1, d2))


def _sc_aggregate(h, send2d, recv2d, eidx, ef1, zA, zD, zE, R, cpt0, cpt1):
    D = h.shape[1]
    DE = 16
    rpt = R // NS
    mesh = plsc.VectorSubcoreMesh(core_axis_name="c", subcore_axis_name="s")

    @functools.partial(
        pl.kernel,
        mesh=mesh,
        out_type=[
            jax.ShapeDtypeStruct((NC * R, D), jnp.float32),
            jax.ShapeDtypeStruct((NC * R * DE,), jnp.float32),
            jax.ShapeDtypeStruct((NC * R,), jnp.float32),
        ],
        scratch_types=[
            pltpu.VMEM((GRP, CHUNK), jnp.int32),
            pltpu.VMEM((GRP, CHUNK), jnp.int32),
            pltpu.VMEM((2, CHUNK, D), jnp.float32),
            pltpu.VMEM((2, CHUNK * DE // 128, 128), jnp.int32),
            pltpu.VMEM((2, CHUNK * DE), jnp.float32),
            pltpu.VMEM((CHUNK,), jnp.float32),
            pltpu.VMEM_SHARED((R, D), jnp.float32),
            pltpu.VMEM_SHARED((R * DE,), jnp.float32),
            pltpu.VMEM_SHARED((R,), jnp.float32),
            pltpu.SemaphoreType.DMA,
            pltpu.SemaphoreType.DMA,
            pltpu.SemaphoreType.DMA,
            pltpu.SemaphoreType.DMA,
            pltpu.SemaphoreType.DMA,
            pltpu.SemaphoreType.DMA,
            pltpu.SemaphoreType.DMA,
            pltpu.SemaphoreType.DMA,
        ],
    )
    def k(h_hbm, s_hbm, r_hbm, eidx_hbm, ef1_hbm, zA_hbm, zD_hbm, zE_hbm,
          outH, outEF, outD,
          idx_s, idx_r, hbuf, eidx_v, efb1, ones64, accH, accEF, accD,
          sg0, sg1, sx0, sx1, sf0, sf1, ss0, ss1):
        c = lax.axis_index("c")
        s = lax.axis_index("s")
        w = c * NS + s
        nsub = CHUNK * DE // 128
        sg = (sg0, sg1)
        sx = (sx0, sx1)
        sf = (sf0, sf1)
        ss = (ss0, ss1)

        pltpu.sync_copy(zA_hbm, accH.at[pl.ds(s * rpt, rpt)])
        pltpu.sync_copy(zD_hbm, accD.at[pl.ds(s * rpt, rpt)])
        pltpu.sync_copy(zE_hbm, accEF.at[pl.ds(s * rpt * DE, rpt * DE)])

        def ob(u, carry):
            ones64[pl.ds(u * 16, 16)] = jnp.ones((16,), jnp.float32)
            return carry

        lax.fori_loop(0, CHUNK // 16, ob, 0)
        plsc.subcore_barrier()

        def loads(g0, j, b):
            j0 = g0 + j
            return (
                pltpu.make_async_copy(h_hbm.at[idx_s.at[j]],
                                      hbuf.at[b], sg[b]),
                pltpu.make_async_copy(eidx_hbm.at[pl.ds(j0 * nsub, nsub)],
                                      eidx_v.at[b], sx[b]),
                pltpu.make_async_copy(
                    ef1_hbm.at[pl.ds(j0 * CHUNK * DE, CHUNK * DE)],
                    efb1.at[b], sf[b]),
            )

        def start(g0, j, b):
            for cp in loads(g0, j, b):
                cp.start()

        def scatter_start(j, b):
            pltpu.async_copy(hbuf.at[b], accH.at[idx_r.at[j]], ss[b],
                             add=True)
            for u in range(nsub):
                pltpu.async_copy(efb1.at[b].at[pl.ds(u * 128, 128)],
                                 accEF.at[eidx_v.at[b].at[u]], ss[b],
                                 add=True)
            pltpu.async_copy(ones64, accD.at[idx_r.at[j]], ss[b], add=True)

        def scatter_drain(b):
            pltpu.make_async_copy(hbuf.at[b], accH.at[idx_r.at[0]],
                                  ss[b]).wait()
            pltpu.make_async_copy(efb1.at[b], accEF.at[pl.ds(0, CHUNK * DE)],
                                  ss[b]).wait()
            pltpu.make_async_copy(ones64, accD.at[pl.ds(0, CHUNK)],
                                  ss[b]).wait()

        base = w * cpt0
        ngrp = cpt0 // GRP

        def group(g, carry):
            g0 = base + g * GRP
            pltpu.sync_copy(s_hbm.at[pl.ds(g0, GRP)], idx_s)
            pltpu.sync_copy(r_hbm.at[pl.ds(g0, GRP)], idx_r)
            start(g0, 0, 0)

            def pair(p, carry2):
                for b in range(2):
                    j = 2 * p + b
                    for cp in loads(g0, j, b):
                        cp.wait()
                    scatter_start(j, b)
                    if b == 0:
                        @pl.when((g > 0) | (p > 0))
                        def _():
                            scatter_drain(1)
                    else:
                        scatter_drain(0)

                    @pl.when(j < GRP - 1)
                    def _():
                        start(g0, j + 1, 1 - b)
                return carry2

            lax.fori_loop(0, GRP // 2, pair, 0)
            return carry

        lax.fori_loop(0, ngrp, group, 0)
        scatter_drain(1)
        plsc.subcore_barrier()

        r0 = s * rpt
        pltpu.sync_copy(accH.at[pl.ds(r0, rpt)],
                        outH.at[pl.ds(c * R + r0, rpt)])
        pltpu.sync_copy(accEF.at[pl.ds(r0 * DE, rpt * DE)],
                        outEF.at[pl.ds((c * R + r0) * DE, rpt * DE)])
        pltpu.sync_copy(accD.at[pl.ds(r0, rpt)],
                        outD.at[pl.ds(c * R + r0, rpt)])

    return k(h, send2d, recv2d, eidx, ef1, zA, zD, zE)


def _combine(h_self, aggH, aggEF, aggD, We, Web):
    n, d = h_self.shape
    de = We.shape[0]
    blk = 2000
    grid = n // blk

    def body(hs_ref, aH_ref, aE_ref, aD_ref, we_ref, wb_ref, out_ref):
        deg = aD_ref[0] + aD_ref[1]
        aggh = aH_ref[0] + aH_ref[1]
        aggef = aE_ref[0] + aE_ref[1]
        t = aggh + jnp.dot(aggef, we_ref[...],
                           preferred_element_type=jnp.float32)
        t = t + deg * wb_ref[...]
        out_ref[...] = hs_ref[...] + jnp.maximum(deg, 1.0) * t

    return pl.pallas_call(
        body,
        grid=(grid,),
        in_specs=[
            pl.BlockSpec((blk, d), lambda i: (i, 0)),
            pl.BlockSpec((2, blk, d), lambda i: (0, i, 0)),
            pl.BlockSpec((2, blk, de), lambda i: (0, i, 0)),
            pl.BlockSpec((2, blk, 1), lambda i: (0, i, 0)),
            pl.BlockSpec((de, d), lambda i: (0, 0)),
            pl.BlockSpec((1, d), lambda i: (0, 0)),
        ],
        out_specs=pl.BlockSpec((blk, d), lambda i: (i, 0)),
        out_shape=jax.ShapeDtypeStruct((n, d), jnp.float32),
    )(h_self, aggH, aggEF, aggD, We, Web.reshape(1, d))


@jax.jit
def kernel(node_features, senders, receivers, edge_features,
           W_kernel, W_bias, We_kernel, We_bias):
    n, d = node_features.shape
    e = senders.shape[0]
    de = edge_features.shape[1]

    cpt = -(-(-(-e // (NW * CHUNK))) // GRP) * GRP
    cpt0 = cpt1 = cpt
    e_pad = NW * cpt * CHUNK
    pad = e_pad - e
    R = -(-(n + 1) // (NS * 64)) * (NS * 64)
    s32 = jnp.concatenate(
        [senders.astype(jnp.int32), jnp.zeros((pad,), jnp.int32)])
    r32 = jnp.concatenate(
        [receivers.astype(jnp.int32), jnp.full((pad,), n, jnp.int32)])
    send2d = s32.reshape(e_pad // CHUNK, CHUNK)
    recv2d = r32.reshape(e_pad // CHUNK, CHUNK)
    eidx = (r32[:, None] * de +
            jnp.arange(de, dtype=jnp.int32)[None, :]).reshape(-1, 128)
    ef1 = jnp.concatenate(
        [edge_features.reshape(-1), jnp.zeros((pad * de,), jnp.float32)])

    rpt = R // NS
    zA = jnp.zeros((rpt, d), jnp.float32)
    zD = jnp.zeros((rpt,), jnp.float32)
    zE = jnp.zeros((rpt * de,), jnp.float32)

    h_self, h = _node_matmul(node_features, W_kernel, W_bias)
    outH, outEF, outD = _sc_aggregate(
        h, send2d, recv2d, eidx, ef1, zA, zD, zE, R, cpt0, cpt1)
    aggH = outH.reshape(NC, R, d)
    aggEF = outEF.reshape(NC, R, de)
    aggD = outD.reshape(NC, R, 1)
    return _combine(h_self, aggH, aggEF, aggD, We_kernel, We_bias)


# speedup vs baseline: 1.2558x; 1.0097x over previous
import functools

import jax
import jax.numpy as jnp
from jax import lax
from jax.experimental import pallas as pl
from jax.experimental.pallas import tpu as pltpu
from jax.experimental.pallas import tpu_sc as plsc

NC = 2
NS = 16
NW = NC * NS
CHUNK = 64
GRP = 16


def _node_matmul(nf, W, Wb):
    n, d = nf.shape
    d2 = W.shape[1]
    blk = 2000
    grid = n // blk

    def body(nf_ref, w_ref, b_ref, hs_ref, h_ref):
        hall = jnp.dot(nf_ref[...], w_ref[...],
                       preferred_element_type=jnp.float32) + b_ref[...]
        hs_ref[...] = hall[:, : d2 // 2]
        h_ref[...] = hall[:, d2 // 2:]

    return pl.pallas_call(
        body,
        grid=(grid,),
        in_specs=[
            pl.BlockSpec((blk, d), lambda i: (i, 0)),
            pl.BlockSpec((d, d2), lambda i: (0, 0)),
            pl.BlockSpec((1, d2), lambda i: (0, 0)),
        ],
        out_specs=[
            pl.BlockSpec((blk, d2 // 2), lambda i: (i, 0)),
            pl.BlockSpec((blk, d2 // 2), lambda i: (i, 0)),
        ],
        out_shape=[
            jax.ShapeDtypeStruct((n, d2 // 2), jnp.float32),
            jax.ShapeDtypeStruct((n, d2 // 2), jnp.float32),
        ],
    )(nf, W, Wb.reshape(1, d2))


def _sc_aggregate(h, send2d, recv2d, eidx, ef1, zA, zD, zE, R, cpt0, cpt1):
    D = h.shape[1]
    DE = 16
    rpt = R // NS
    mesh = plsc.VectorSubcoreMesh(core_axis_name="c", subcore_axis_name="s")

    @functools.partial(
        pl.kernel,
        mesh=mesh,
        out_type=[
            jax.ShapeDtypeStruct((NC * R, D), jnp.float32),
            jax.ShapeDtypeStruct((NC * R * DE,), jnp.float32),
            jax.ShapeDtypeStruct((NC * R,), jnp.float32),
        ],
        scratch_types=[
            pltpu.VMEM((GRP, CHUNK), jnp.int32),
            pltpu.VMEM((GRP, CHUNK), jnp.int32),
            pltpu.VMEM((2, CHUNK, D), jnp.float32),
            pltpu.VMEM((2, CHUNK * DE // 128, 128), jnp.int32),
            pltpu.VMEM((2, CHUNK * DE), jnp.float32),
            pltpu.VMEM((CHUNK,), jnp.float32),
            pltpu.VMEM_SHARED((R, D), jnp.float32),
            pltpu.VMEM_SHARED((R * DE,), jnp.float32),
            pltpu.VMEM_SHARED((R,), jnp.float32),
            pltpu.SemaphoreType.DMA,
            pltpu.SemaphoreType.DMA,
            pltpu.SemaphoreType.DMA,
            pltpu.SemaphoreType.DMA,
            pltpu.SemaphoreType.DMA,
            pltpu.SemaphoreType.DMA,
            pltpu.SemaphoreType.DMA,
            pltpu.SemaphoreType.DMA,
        ],
    )
    def k(h_hbm, s_hbm, r_hbm, eidx_hbm, ef1_hbm, zA_hbm, zD_hbm, zE_hbm,
          outH, outEF, outD,
          idx_s, idx_r, hbuf, eidx_v, efb1, ones64, accH, accEF, accD,
          sg0, sg1, sx0, sx1, sf0, sf1, ss0, ss1):
        c = lax.axis_index("c")
        s = lax.axis_index("s")
        w = c * NS + s
        nsub = CHUNK * DE // 128
        sg = (sg0, sg1)
        sx = (sx0, sx1)
        sf = (sf0, sf1)
        ss = (ss0, ss1)

        pltpu.sync_copy(zA_hbm, accH.at[pl.ds(s * rpt, rpt)])
        pltpu.sync_copy(zD_hbm, accD.at[pl.ds(s * rpt, rpt)])
        pltpu.sync_copy(zE_hbm, accEF.at[pl.ds(s * rpt * DE, rpt * DE)])

        def ob(u, carry):
            ones64[pl.ds(u * 16, 16)] = jnp.ones((16,), jnp.float32)
            return carry

        lax.fori_loop(0, CHUNK // 16, ob, 0)
        plsc.subcore_barrier()

        def loads(g0, j, b):
            j0 = g0 + j
            return (
                pltpu.make_async_copy(h_hbm.at[idx_s.at[j]],
                                      hbuf.at[b], sg[b]),
                pltpu.make_async_copy(eidx_hbm.at[pl.ds(j0 * nsub, nsub)],
                                      eidx_v.at[b], sx[b]),
                pltpu.make_async_copy(
                    ef1_hbm.at[pl.ds(j0 * CHUNK * DE, CHUNK * DE)],
                    efb1.at[b], sf[b]),
            )

        def start(g0, j, b):
            for cp in loads(g0, j, b):
                cp.start()

        def scatter_start(j, b):
            pltpu.async_copy(hbuf.at[b], accH.at[idx_r.at[j]], ss[b],
                             add=True)
            for u in range(nsub):
                pltpu.async_copy(efb1.at[b].at[pl.ds(u * 128, 128)],
                                 accEF.at[eidx_v.at[b].at[u]], ss[b],
                                 add=True)
            pltpu.async_copy(ones64, accD.at[idx_r.at[j]], ss[b], add=True)

        def scatter_drain(b):
            pltpu.make_async_copy(hbuf.at[b], accH.at[idx_r.at[0]],
                                  ss[b]).wait()
            pltpu.make_async_copy(efb1.at[b], accEF.at[pl.ds(0, CHUNK * DE)],
                                  ss[b]).wait()
            pltpu.make_async_copy(ones64, accD.at[pl.ds(0, CHUNK)],
                                  ss[b]).wait()

        base = w * cpt0
        ngrp = cpt0 // GRP

        def group(g, carry):
            g0 = base + g * GRP
            pltpu.sync_copy(s_hbm.at[pl.ds(g0, GRP)], idx_s)
            pltpu.sync_copy(r_hbm.at[pl.ds(g0, GRP)], idx_r)
            start(g0, 0, 0)

            def pair(p, carry2):
                for b in range(2):
                    j = 2 * p + b
                    for cp in loads(g0, j, b):
                        cp.wait()
                    scatter_start(j, b)
                    if b == 0:
                        @pl.when((g > 0) | (p > 0))
                        def _():
                            scatter_drain(1)
                    else:
                        scatter_drain(0)

                    @pl.when(j < GRP - 1)
                    def _():
                        start(g0, j + 1, 1 - b)
                return carry2

            lax.fori_loop(0, GRP // 2, pair, 0)
            return carry

        lax.fori_loop(0, ngrp, group, 0)
        scatter_drain(1)
        plsc.subcore_barrier()

        r0 = s * rpt
        pltpu.sync_copy(accH.at[pl.ds(r0, rpt)],
                        outH.at[pl.ds(c * R + r0, rpt)])
        pltpu.sync_copy(accEF.at[pl.ds(r0 * DE, rpt * DE)],
                        outEF.at[pl.ds((c * R + r0) * DE, rpt * DE)])
        pltpu.sync_copy(accD.at[pl.ds(r0, rpt)],
                        outD.at[pl.ds(c * R + r0, rpt)])

    return k(h, send2d, recv2d, eidx, ef1, zA, zD, zE)


def _combine(h_self, aggH, aggEF, aggD, We, Web):
    n, d = h_self.shape
    de = We.shape[0]
    blk = 2000
    grid = n // blk

    def body(hs_ref, aH_ref, aE_ref, aD_ref, we_ref, wb_ref, out_ref):
        deg = aD_ref[0] + aD_ref[1]
        aggh = aH_ref[0] + aH_ref[1]
        aggef = aE_ref[0] + aE_ref[1]
        t = aggh + jnp.dot(aggef, we_ref[...],
                           preferred_element_type=jnp.float32)
        t = t + deg * wb_ref[...]
        out_ref[...] = hs_ref[...] + jnp.maximum(deg, 1.0) * t

    return pl.pallas_call(
        body,
        grid=(grid,),
        in_specs=[
            pl.BlockSpec((blk, d), lambda i: (i, 0)),
            pl.BlockSpec((2, blk, d), lambda i: (0, i, 0)),
            pl.BlockSpec((2, blk, de), lambda i: (0, i, 0)),
            pl.BlockSpec((2, blk, 1), lambda i: (0, i, 0)),
            pl.BlockSpec((de, d), lambda i: (0, 0)),
            pl.BlockSpec((1, d), lambda i: (0, 0)),
        ],
        out_specs=pl.BlockSpec((blk, d), lambda i: (i, 0)),
        out_shape=jax.ShapeDtypeStruct((n, d), jnp.float32),
    )(h_self, aggH, aggEF, aggD, We, Web.reshape(1, d))


@jax.jit
def kernel(node_features, senders, receivers, edge_features,
           W_kernel, W_bias, We_kernel, We_bias):
    n, d = node_features.shape
    e = senders.shape[0]
    de = edge_features.shape[1]

    cpt = -(-(-(-e // (NW * CHUNK))) // GRP) * GRP
    cpt0 = cpt1 = cpt
    e_pad = NW * cpt * CHUNK
    pad = e_pad - e
    R = -(-(n + 1) // (NS * 64)) * (NS * 64)
    s32 = jnp.concatenate(
        [senders.astype(jnp.int32), jnp.zeros((pad,), jnp.int32)])
    r32 = jnp.concatenate(
        [receivers.astype(jnp.int32), jnp.full((pad,), n, jnp.int32)])
    send2d = s32.reshape(e_pad // CHUNK, CHUNK)
    recv2d = r32.reshape(e_pad // CHUNK, CHUNK)
    eidx = (r32[:, None] * de +
            jnp.arange(de, dtype=jnp.int32)[None, :]).reshape(-1, 128)
    ef1 = jnp.concatenate(
        [edge_features.reshape(-1), jnp.zeros((pad * de,), jnp.float32)])

    rpt = R // NS
    zA = jnp.zeros((rpt, d), jnp.float32)
    zD = jnp.zeros((rpt,), jnp.float32)
    zE = jnp.zeros((rpt * de,), jnp.float32)

    h_self, h = _node_matmul(node_features, W_kernel, W_bias)
    outH, outEF, outD = _sc_aggregate(
        h, send2d, recv2d, eidx, ef1, zA, zD, zE, R, cpt0, cpt1)
    aggH = outH.reshape(NC, R, d)
    aggEF = outEF.reshape(NC, R, de)
    aggD = outD.reshape(NC, R, 1)
    return _combine(h_self, aggH, aggEF, aggD, We_kernel, We_bias)


# speedup vs baseline: 1.2633x; 1.0059x over previous
import functools

import jax
import jax.numpy as jnp
from jax import lax
from jax.experimental import pallas as pl
from jax.experimental.pallas import tpu as pltpu
from jax.experimental.pallas import tpu_sc as plsc

NC = 2
NS = 16
NW = NC * NS
CHUNK = 64
GRP = 32


def _node_matmul(nf, W, Wb):
    n, d = nf.shape
    d2 = W.shape[1]
    blk = 2000
    grid = n // blk

    def body(nf_ref, w_ref, b_ref, hs_ref, h_ref):
        hall = jnp.dot(nf_ref[...], w_ref[...],
                       preferred_element_type=jnp.float32) + b_ref[...]
        hs_ref[...] = hall[:, : d2 // 2]
        h_ref[...] = hall[:, d2 // 2:]

    return pl.pallas_call(
        body,
        grid=(grid,),
        in_specs=[
            pl.BlockSpec((blk, d), lambda i: (i, 0)),
            pl.BlockSpec((d, d2), lambda i: (0, 0)),
            pl.BlockSpec((1, d2), lambda i: (0, 0)),
        ],
        out_specs=[
            pl.BlockSpec((blk, d2 // 2), lambda i: (i, 0)),
            pl.BlockSpec((blk, d2 // 2), lambda i: (i, 0)),
        ],
        out_shape=[
            jax.ShapeDtypeStruct((n, d2 // 2), jnp.float32),
            jax.ShapeDtypeStruct((n, d2 // 2), jnp.float32),
        ],
    )(nf, W, Wb.reshape(1, d2))


def _sc_aggregate(h, send2d, recv2d, eidx, ef1, zA, zD, zE, R, cpt0, cpt1):
    D = h.shape[1]
    DE = 16
    rpt = R // NS
    mesh = plsc.VectorSubcoreMesh(core_axis_name="c", subcore_axis_name="s")

    @functools.partial(
        pl.kernel,
        mesh=mesh,
        out_type=[
            jax.ShapeDtypeStruct((NC * R, D), jnp.float32),
            jax.ShapeDtypeStruct((NC * R * DE,), jnp.float32),
            jax.ShapeDtypeStruct((NC * R,), jnp.float32),
        ],
        scratch_types=[
            pltpu.VMEM((GRP, CHUNK), jnp.int32),
            pltpu.VMEM((GRP, CHUNK), jnp.int32),
            pltpu.VMEM((2, CHUNK, D), jnp.float32),
            pltpu.VMEM((2, CHUNK * DE // 128, 128), jnp.int32),
            pltpu.VMEM((2, CHUNK * DE), jnp.float32),
            pltpu.VMEM((CHUNK,), jnp.float32),
            pltpu.VMEM_SHARED((R, D), jnp.float32),
            pltpu.VMEM_SHARED((R * DE,), jnp.float32),
            pltpu.VMEM_SHARED((R,), jnp.float32),
            pltpu.SemaphoreType.DMA,
            pltpu.SemaphoreType.DMA,
            pltpu.SemaphoreType.DMA,
            pltpu.SemaphoreType.DMA,
            pltpu.SemaphoreType.DMA,
            pltpu.SemaphoreType.DMA,
            pltpu.SemaphoreType.DMA,
            pltpu.SemaphoreType.DMA,
        ],
    )
    def k(h_hbm, s_hbm, r_hbm, eidx_hbm, ef1_hbm, zA_hbm, zD_hbm, zE_hbm,
          outH, outEF, outD,
          idx_s, idx_r, hbuf, eidx_v, efb1, ones64, accH, accEF, accD,
          sg0, sg1, sx0, sx1, sf0, sf1, ss0, ss1):
        c = lax.axis_index("c")
        s = lax.axis_index("s")
        w = c * NS + s
        nsub = CHUNK * DE // 128
        sg = (sg0, sg1)
        sx = (sx0, sx1)
        sf = (sf0, sf1)
        ss = (ss0, ss1)

        pltpu.sync_copy(zA_hbm, accH.at[pl.ds(s * rpt, rpt)])
        pltpu.sync_copy(zD_hbm, accD.at[pl.ds(s * rpt, rpt)])
        pltpu.sync_copy(zE_hbm, accEF.at[pl.ds(s * rpt * DE, rpt * DE)])

        def ob(u, carry):
            ones64[pl.ds(u * 16, 16)] = jnp.ones((16,), jnp.float32)
            return carry

        lax.fori_loop(0, CHUNK // 16, ob, 0)
        plsc.subcore_barrier()

        def loads(g0, j, b):
            j0 = g0 + j
            return (
                pltpu.make_async_copy(h_hbm.at[idx_s.at[j]],
                                      hbuf.at[b], sg[b]),
                pltpu.make_async_copy(eidx_hbm.at[pl.ds(j0 * nsub, nsub)],
                                      eidx_v.at[b], sx[b]),
                pltpu.make_async_copy(
                    ef1_hbm.at[pl.ds(j0 * CHUNK * DE, CHUNK * DE)],
                    efb1.at[b], sf[b]),
            )

        def start(g0, j, b):
            for cp in loads(g0, j, b):
                cp.start()

        def scatter_start(j, b):
            pltpu.async_copy(hbuf.at[b], accH.at[idx_r.at[j]], ss[b],
                             add=True)
            for u in range(nsub):
                pltpu.async_copy(efb1.at[b].at[pl.ds(u * 128, 128)],
                                 accEF.at[eidx_v.at[b].at[u]], ss[b],
                                 add=True)
            pltpu.async_copy(ones64, accD.at[idx_r.at[j]], ss[b], add=True)

        def scatter_drain(b):
            pltpu.make_async_copy(hbuf.at[b], accH.at[idx_r.at[0]],
                                  ss[b]).wait()
            pltpu.make_async_copy(efb1.at[b], accEF.at[pl.ds(0, CHUNK * DE)],
                                  ss[b]).wait()
            pltpu.make_async_copy(ones64, accD.at[pl.ds(0, CHUNK)],
                                  ss[b]).wait()

        base = w * cpt0
        ngrp = cpt0 // GRP

        def group(g, carry):
            g0 = base + g * GRP
            pltpu.sync_copy(s_hbm.at[pl.ds(g0, GRP)], idx_s)
            pltpu.sync_copy(r_hbm.at[pl.ds(g0, GRP)], idx_r)
            start(g0, 0, 0)

            def pair(p, carry2):
                for b in range(2):
                    j = 2 * p + b
                    for cp in loads(g0, j, b):
                        cp.wait()
                    scatter_start(j, b)
                    if b == 0:
                        @pl.when((g > 0) | (p > 0))
                        def _():
                            scatter_drain(1)
                    else:
                        scatter_drain(0)

                    @pl.when(j < GRP - 1)
                    def _():
                        start(g0, j + 1, 1 - b)
                return carry2

            lax.fori_loop(0, GRP // 2, pair, 0)
            return carry

        lax.fori_loop(0, ngrp, group, 0)
        scatter_drain(1)
        plsc.subcore_barrier()

        r0 = s * rpt
        pltpu.sync_copy(accH.at[pl.ds(r0, rpt)],
                        outH.at[pl.ds(c * R + r0, rpt)])
        pltpu.sync_copy(accEF.at[pl.ds(r0 * DE, rpt * DE)],
                        outEF.at[pl.ds((c * R + r0) * DE, rpt * DE)])
        pltpu.sync_copy(accD.at[pl.ds(r0, rpt)],
                        outD.at[pl.ds(c * R + r0, rpt)])

    return k(h, send2d, recv2d, eidx, ef1, zA, zD, zE)


def _combine(h_self, aggH, aggEF, aggD, We, Web):
    n, d = h_self.shape
    de = We.shape[0]
    blk = 2000
    grid = n // blk

    def body(hs_ref, aH_ref, aE_ref, aD_ref, we_ref, wb_ref, out_ref):
        deg = aD_ref[0] + aD_ref[1]
        aggh = aH_ref[0] + aH_ref[1]
        aggef = aE_ref[0] + aE_ref[1]
        t = aggh + jnp.dot(aggef, we_ref[...],
                           preferred_element_type=jnp.float32)
        t = t + deg * wb_ref[...]
        out_ref[...] = hs_ref[...] + jnp.maximum(deg, 1.0) * t

    return pl.pallas_call(
        body,
        grid=(grid,),
        in_specs=[
            pl.BlockSpec((blk, d), lambda i: (i, 0)),
            pl.BlockSpec((2, blk, d), lambda i: (0, i, 0)),
            pl.BlockSpec((2, blk, de), lambda i: (0, i, 0)),
            pl.BlockSpec((2, blk, 1), lambda i: (0, i, 0)),
            pl.BlockSpec((de, d), lambda i: (0, 0)),
            pl.BlockSpec((1, d), lambda i: (0, 0)),
        ],
        out_specs=pl.BlockSpec((blk, d), lambda i: (i, 0)),
        out_shape=jax.ShapeDtypeStruct((n, d), jnp.float32),
    )(h_self, aggH, aggEF, aggD, We, Web.reshape(1, d))


@jax.jit
def kernel(node_features, senders, receivers, edge_features,
           W_kernel, W_bias, We_kernel, We_bias):
    n, d = node_features.shape
    e = senders.shape[0]
    de = edge_features.shape[1]

    cpt = -(-(-(-e // (NW * CHUNK))) // GRP) * GRP
    cpt0 = cpt1 = cpt
    e_pad = NW * cpt * CHUNK
    pad = e_pad - e
    R = -(-(n + 1) // (NS * 64)) * (NS * 64)
    s32 = jnp.concatenate(
        [senders.astype(jnp.int32), jnp.zeros((pad,), jnp.int32)])
    r32 = jnp.concatenate(
        [receivers.astype(jnp.int32), jnp.full((pad,), n, jnp.int32)])
    send2d = s32.reshape(e_pad // CHUNK, CHUNK)
    recv2d = r32.reshape(e_pad // CHUNK, CHUNK)
    eidx = (r32[:, None] * de +
            jnp.arange(de, dtype=jnp.int32)[None, :]).reshape(-1, 128)
    ef1 = jnp.concatenate(
        [edge_features.reshape(-1), jnp.zeros((pad * de,), jnp.float32)])

    rpt = R // NS
    zA = jnp.zeros((rpt, d), jnp.float32)
    zD = jnp.zeros((rpt,), jnp.float32)
    zE = jnp.zeros((rpt * de,), jnp.float32)

    h_self, h = _node_matmul(node_features, W_kernel, W_bias)
    outH, outEF, outD = _sc_aggregate(
        h, send2d, recv2d, eidx, ef1, zA, zD, zE, R, cpt0, cpt1)
    aggH = outH.reshape(NC, R, d)
    aggEF = outEF.reshape(NC, R, de)
    aggD = outD.reshape(NC, R, 1)
    return _combine(h_self, aggH, aggEF, aggD, We_kernel, We_bias)
